# baseline (speedup 1.0000x reference)
import jax
import jax.numpy as jnp
import numpy as np
from jax.experimental import pallas as pl
from jax.experimental.pallas import tpu as pltpu

_H = 14
_LH = 10
_BB = 128


def _fused(x_ref, w0_ref, gt_ref, wp_ref, o_ref, nw_ref):
    H, LH = _H, _LH
    out_u, in_u = gt_ref.shape[1], gt_ref.shape[2]
    inv_in = np.float32(1.0 / (in_u - 1))
    inv_out = np.float32(1.0 / (out_u - 1))
    f32 = jnp.float32

    @pl.when(pl.program_id(0) == 0)
    def _mlp():
        gt = gt_ref[...]
        wT = jnp.transpose(w0_ref[...])

        csh = jnp.sum(gt, axis=2)
        rsh = jnp.sum(gt, axis=1)
        csw = jnp.sum(wT, axis=1, keepdims=True)
        rsw = jnp.sum(wT, axis=0, keepdims=True)

        A = wp_ref[3:3 + H, :]
        B = wp_ref[3 + H:3 + 2 * H, :]
        C = wp_ref[3 + 2 * H:3 + 3 * H, :]
        meffT = jnp.transpose(A - B * inv_in - C * inv_out)

        dn = (((1,), (0,)), ((), ()))
        core = jax.lax.dot_general(meffT, gt, dn,
                                   preferred_element_type=f32)

        colcorr = (jnp.dot(jnp.transpose(B), csh,
                           preferred_element_type=f32) * inv_in
                   + jnp.transpose(csw * (wp_ref[1:2, :] * inv_in)))
        rowcorr = (jnp.dot(jnp.transpose(C), rsh,
                           preferred_element_type=f32) * inv_out
                   + jnp.transpose(wp_ref[2:3, :]) * (rsw * inv_out))
        weffT = jnp.transpose(wp_ref[0:1, :] - wp_ref[1:2, :] * inv_in
                              - wp_ref[2:3, :] * inv_out)
        b1T = jnp.transpose(wp_ref[45:46, :])

        pre1 = (core + wT[None, :, :] * weffT[:, :, None]
                + colcorr[:, :, None] + rowcorr[:, None, :]
                + b1T[:, :, None])
        a1 = jnp.maximum(pre1, 0.0)

        pre2 = jax.lax.dot_general(jnp.transpose(wp_ref[46:56, :]), a1, dn,
                                   preferred_element_type=f32)
        a2 = jnp.maximum(pre2 + jnp.transpose(wp_ref[56:57, :])[:, :, None],
                         0.0)

        w3T = jnp.transpose(wp_ref[57:67, 0:1])
        updT = jax.lax.dot_general(w3T, a2, dn,
                                   preferred_element_type=f32)[0]
        new_wT = wT + updT + wp_ref[67:68, 0:1]
        nw_ref[...] = jnp.transpose(new_wT)

    logits = jnp.maximum(
        jnp.dot(x_ref[...], nw_ref[...], preferred_element_type=f32), 0.0)
    m = jnp.max(logits, axis=1, keepdims=True)
    e = jnp.exp(logits - m)
    o_ref[...] = e / jnp.sum(e, axis=1, keepdims=True)


def kernel(X, w0, h0, W1, b1, W2, b2, W3, b3):
    in_u, out_u, H = h0.shape
    batch = X.shape[0]
    gt = h0.transpose(2, 1, 0)
    LH = W2.shape[0]
    wpack = jnp.concatenate([
        W1, b1[None, :], W2, b2[None, :],
        jnp.pad(W3[:, 0:1], ((0, 0), (0, LH - 1))),
        jnp.pad(b3[0:1][None, :], ((0, 0), (0, LH - 1))),
    ], axis=0)

    vmem = pltpu.VMEM
    grid = batch // _BB
    return pl.pallas_call(
        _fused,
        grid=(grid,),
        out_shape=jax.ShapeDtypeStruct((batch, out_u), jnp.float32),
        in_specs=[
            pl.BlockSpec((_BB, in_u), lambda i: (i, 0)),
            pl.BlockSpec((in_u, out_u), lambda i: (0, 0)),
            pl.BlockSpec((H, out_u, in_u), lambda i: (0, 0, 0)),
            pl.BlockSpec((68, _LH), lambda i: (0, 0)),
        ],
        out_specs=pl.BlockSpec((_BB, out_u), lambda i: (i, 0)),
        scratch_shapes=[vmem((in_u, out_u), jnp.float32)],
    )(X, w0, gt, wpack)


# speedup vs baseline: 1.6536x; 1.6536x over previous
import jax
import jax.numpy as jnp
import numpy as np
from jax.experimental import pallas as pl
from jax.experimental.pallas import tpu as pltpu

_H = 14
_LH = 10
_BB = 256


def _fused(x_ref, w0_ref, gt_ref, w1v_ref, b1_ref, w2_ref,
           b2_ref, w3_ref, b3_ref, o_ref, nw_ref):
    H, LH = _H, _LH
    out_u, in_u = gt_ref.shape[1], gt_ref.shape[2]
    inv_in = np.float32(1.0 / (in_u - 1))
    inv_out = np.float32(1.0 / (out_u - 1))
    f32 = jnp.float32

    @pl.when(pl.program_id(0) == 0)
    def _mlp():
        gt = gt_ref[...]
        wT = jnp.transpose(w0_ref[...])

        csh = jnp.sum(gt, axis=2)
        rsh = jnp.sum(gt, axis=1)
        csw = jnp.sum(wT, axis=1, keepdims=True)
        rsw = jnp.sum(wT, axis=0, keepdims=True)

        A = w1v_ref[3:3 + H, :]
        B = w1v_ref[3 + H:3 + 2 * H, :]
        C = w1v_ref[3 + 2 * H:3 + 3 * H, :]
        meffT = jnp.transpose(A - B * inv_in - C * inv_out)

        dn = (((1,), (0,)), ((), ()))
        core = jax.lax.dot_general(meffT, gt, dn,
                                   preferred_element_type=f32)

        colcorr = (jnp.dot(jnp.transpose(B), csh,
                           preferred_element_type=f32) * inv_in
                   + jnp.transpose(csw * (w1v_ref[1:2, :] * inv_in)))
        rowcorr = (jnp.dot(jnp.transpose(C), rsh,
                           preferred_element_type=f32) * inv_out
                   + jnp.transpose(w1v_ref[2:3, :]) * (rsw * inv_out))
        weffT = jnp.transpose(w1v_ref[0:1, :] - w1v_ref[1:2, :] * inv_in
                              - w1v_ref[2:3, :] * inv_out)
        b1T = jnp.transpose(b1_ref[...])

        pre1 = (core + wT[None, :, :] * weffT[:, :, None]
                + colcorr[:, :, None] + rowcorr[:, None, :]
                + b1T[:, :, None])
        a1 = jnp.maximum(pre1, 0.0)

        pre2 = jax.lax.dot_general(jnp.transpose(w2_ref[...]), a1, dn,
                                   preferred_element_type=f32)
        a2 = jnp.maximum(pre2 + jnp.transpose(b2_ref[...])[:, :, None],
                         0.0)

        w3T = jnp.transpose(w3_ref[...][:, 0:1])
        updT = jax.lax.dot_general(w3T, a2, dn,
                                   preferred_element_type=f32)[0]
        new_wT = wT + updT + b3_ref[0, 0]
        nw_ref[...] = jnp.transpose(new_wT)

    logits = jnp.maximum(
        jnp.dot(x_ref[...], nw_ref[...], preferred_element_type=f32), 0.0)
    m = jnp.max(logits, axis=1, keepdims=True)
    e = jnp.exp(logits - m)
    o_ref[...] = e / jnp.sum(e, axis=1, keepdims=True)


def kernel(X, w0, h0, W1, b1, W2, b2, W3, b3):
    in_u, out_u, H = h0.shape
    batch = X.shape[0]
    gt = h0.transpose(2, 1, 0)

    vmem = pltpu.VMEM
    grid = batch // _BB
    return pl.pallas_call(
        _fused,
        grid=(grid,),
        out_shape=jax.ShapeDtypeStruct((batch, out_u), jnp.float32),
        in_specs=[
            pl.BlockSpec((_BB, in_u), lambda i: (i, 0)),
            pl.BlockSpec((in_u, out_u), lambda i: (0, 0)),
            pl.BlockSpec((H, out_u, in_u), lambda i: (0, 0, 0)),
            pl.BlockSpec((45, _LH), lambda i: (0, 0)),
            pl.BlockSpec((1, _LH), lambda i: (0, 0)),
            pl.BlockSpec((_LH, _LH), lambda i: (0, 0)),
            pl.BlockSpec((1, _LH), lambda i: (0, 0)),
            pl.BlockSpec((_LH, 15), lambda i: (0, 0)),
            pl.BlockSpec(memory_space=pltpu.SMEM),
        ],
        out_specs=pl.BlockSpec((_BB, out_u), lambda i: (i, 0)),
        scratch_shapes=[vmem((in_u, out_u), jnp.float32)],
    )(X, w0, gt, W1,
      b1[None, :], W2, b2[None, :], W3, b3[None, :])


# speedup vs baseline: 1.8926x; 1.1445x over previous
import jax
import jax.numpy as jnp
import numpy as np
from jax.experimental import pallas as pl
from jax.experimental.pallas import tpu as pltpu

_H = 14
_LH = 10
_BB = 512


def _fused(x_ref, w0_ref, gt_ref, w1v_ref, b1_ref, w2_ref,
           b2_ref, w3_ref, b3_ref, o_ref, nw_ref):
    H, LH = _H, _LH
    out_u, in_u = gt_ref.shape[1], gt_ref.shape[2]
    inv_in = np.float32(1.0 / (in_u - 1))
    inv_out = np.float32(1.0 / (out_u - 1))
    f32 = jnp.float32

    @pl.when(pl.program_id(0) == 0)
    def _mlp():
        gt = gt_ref[...]
        wT = jnp.transpose(w0_ref[...])

        csh = jnp.sum(gt, axis=2)
        rsh = jnp.sum(gt, axis=1)
        csw = jnp.sum(wT, axis=1, keepdims=True)
        rsw = jnp.sum(wT, axis=0, keepdims=True)

        A = w1v_ref[3:3 + H, :]
        B = w1v_ref[3 + H:3 + 2 * H, :]
        C = w1v_ref[3 + 2 * H:3 + 3 * H, :]
        meffT = jnp.transpose(A - B * inv_in - C * inv_out)

        dn = (((1,), (0,)), ((), ()))
        core = jax.lax.dot_general(meffT, gt, dn,
                                   preferred_element_type=f32)

        colcorr = (jnp.dot(jnp.transpose(B), csh,
                           preferred_element_type=f32) * inv_in
                   + jnp.transpose(csw * (w1v_ref[1:2, :] * inv_in)))
        rowcorr = (jnp.dot(jnp.transpose(C), rsh,
                           preferred_element_type=f32) * inv_out
                   + jnp.transpose(w1v_ref[2:3, :]) * (rsw * inv_out))
        weffT = jnp.transpose(w1v_ref[0:1, :] - w1v_ref[1:2, :] * inv_in
                              - w1v_ref[2:3, :] * inv_out)
        b1T = jnp.transpose(b1_ref[...])

        pre1 = (core + wT[None, :, :] * weffT[:, :, None]
                + colcorr[:, :, None] + rowcorr[:, None, :]
                + b1T[:, :, None])
        a1 = jnp.maximum(pre1, 0.0)

        pre2 = jax.lax.dot_general(jnp.transpose(w2_ref[...]), a1, dn,
                                   preferred_element_type=f32)
        a2 = jnp.maximum(pre2 + jnp.transpose(b2_ref[...])[:, :, None],
                         0.0)

        w3T = jnp.transpose(w3_ref[...][:, 0:1])
        updT = jax.lax.dot_general(w3T, a2, dn,
                                   preferred_element_type=f32)[0]
        new_wT = wT + updT + b3_ref[0, 0]
        nw_ref[...] = jnp.transpose(new_wT)

    logits = jnp.maximum(
        jnp.dot(x_ref[...], nw_ref[...], preferred_element_type=f32), 0.0)
    m = jnp.max(logits, axis=1, keepdims=True)
    e = jnp.exp(logits - m)
    o_ref[...] = e / jnp.sum(e, axis=1, keepdims=True)


def kernel(X, w0, h0, W1, b1, W2, b2, W3, b3):
    in_u, out_u, H = h0.shape
    batch = X.shape[0]
    gt = h0.transpose(2, 1, 0)

    vmem = pltpu.VMEM
    grid = batch // _BB
    return pl.pallas_call(
        _fused,
        grid=(grid,),
        out_shape=jax.ShapeDtypeStruct((batch, out_u), jnp.float32),
        in_specs=[
            pl.BlockSpec((_BB, in_u), lambda i: (i, 0)),
            pl.BlockSpec((in_u, out_u), lambda i: (0, 0)),
            pl.BlockSpec((H, out_u, in_u), lambda i: (0, 0, 0)),
            pl.BlockSpec((45, _LH), lambda i: (0, 0)),
            pl.BlockSpec((1, _LH), lambda i: (0, 0)),
            pl.BlockSpec((_LH, _LH), lambda i: (0, 0)),
            pl.BlockSpec((1, _LH), lambda i: (0, 0)),
            pl.BlockSpec((_LH, 15), lambda i: (0, 0)),
            pl.BlockSpec(memory_space=pltpu.SMEM),
        ],
        out_specs=pl.BlockSpec((_BB, out_u), lambda i: (i, 0)),
        scratch_shapes=[vmem((in_u, out_u), jnp.float32)],
    )(X, w0, gt, W1,
      b1[None, :], W2, b2[None, :], W3, b3[None, :])


# speedup vs baseline: 1.9269x; 1.0181x over previous
import jax
import jax.numpy as jnp
import numpy as np
from jax.experimental import pallas as pl
from jax.experimental.pallas import tpu as pltpu

_H = 14
_LH = 10
_BB = 1024


def _fused(x_ref, w0_ref, gt_ref, w1v_ref, b1_ref, w2_ref,
           b2_ref, w3_ref, b3_ref, o_ref, nw_ref):
    H, LH = _H, _LH
    out_u, in_u = gt_ref.shape[1], gt_ref.shape[2]
    inv_in = np.float32(1.0 / (in_u - 1))
    inv_out = np.float32(1.0 / (out_u - 1))
    f32 = jnp.float32

    @pl.when(pl.program_id(0) == 0)
    def _mlp():
        gt = gt_ref[...]
        wT = jnp.transpose(w0_ref[...])

        csh = jnp.sum(gt, axis=2)
        rsh = jnp.sum(gt, axis=1)
        csw = jnp.sum(wT, axis=1, keepdims=True)
        rsw = jnp.sum(wT, axis=0, keepdims=True)

        A = w1v_ref[3:3 + H, :]
        B = w1v_ref[3 + H:3 + 2 * H, :]
        C = w1v_ref[3 + 2 * H:3 + 3 * H, :]
        meffT = jnp.transpose(A - B * inv_in - C * inv_out)

        dn = (((1,), (0,)), ((), ()))
        core = jax.lax.dot_general(meffT, gt, dn,
                                   preferred_element_type=f32)

        colcorr = (jnp.dot(jnp.transpose(B), csh,
                           preferred_element_type=f32) * inv_in
                   + jnp.transpose(csw * (w1v_ref[1:2, :] * inv_in)))
        rowcorr = (jnp.dot(jnp.transpose(C), rsh,
                           preferred_element_type=f32) * inv_out
                   + jnp.transpose(w1v_ref[2:3, :]) * (rsw * inv_out))
        weffT = jnp.transpose(w1v_ref[0:1, :] - w1v_ref[1:2, :] * inv_in
                              - w1v_ref[2:3, :] * inv_out)
        b1T = jnp.transpose(b1_ref[...])

        pre1 = (core + wT[None, :, :] * weffT[:, :, None]
                + colcorr[:, :, None] + rowcorr[:, None, :]
                + b1T[:, :, None])
        a1 = jnp.maximum(pre1, 0.0)

        pre2 = jax.lax.dot_general(jnp.transpose(w2_ref[...]), a1, dn,
                                   preferred_element_type=f32)
        a2 = jnp.maximum(pre2 + jnp.transpose(b2_ref[...])[:, :, None],
                         0.0)

        w3T = jnp.transpose(w3_ref[...][:, 0:1])
        updT = jax.lax.dot_general(w3T, a2, dn,
                                   preferred_element_type=f32)[0]
        new_wT = wT + updT + b3_ref[0, 0]
        nw_ref[...] = jnp.transpose(new_wT)

    logits = jnp.maximum(
        jnp.dot(x_ref[...], nw_ref[...], preferred_element_type=f32), 0.0)
    m = jnp.max(logits, axis=1, keepdims=True)
    e = jnp.exp(logits - m)
    o_ref[...] = e / jnp.sum(e, axis=1, keepdims=True)


def kernel(X, w0, h0, W1, b1, W2, b2, W3, b3):
    in_u, out_u, H = h0.shape
    batch = X.shape[0]
    gt = h0.transpose(2, 1, 0)

    vmem = pltpu.VMEM
    grid = batch // _BB
    return pl.pallas_call(
        _fused,
        grid=(grid,),
        out_shape=jax.ShapeDtypeStruct((batch, out_u), jnp.float32),
        in_specs=[
            pl.BlockSpec((_BB, in_u), lambda i: (i, 0)),
            pl.BlockSpec((in_u, out_u), lambda i: (0, 0)),
            pl.BlockSpec((H, out_u, in_u), lambda i: (0, 0, 0)),
            pl.BlockSpec((45, _LH), lambda i: (0, 0)),
            pl.BlockSpec((1, _LH), lambda i: (0, 0)),
            pl.BlockSpec((_LH, _LH), lambda i: (0, 0)),
            pl.BlockSpec((1, _LH), lambda i: (0, 0)),
            pl.BlockSpec((_LH, 15), lambda i: (0, 0)),
            pl.BlockSpec(memory_space=pltpu.SMEM),
        ],
        out_specs=pl.BlockSpec((_BB, out_u), lambda i: (i, 0)),
        scratch_shapes=[vmem((in_u, out_u), jnp.float32)],
    )(X, w0, gt, W1,
      b1[None, :], W2, b2[None, :], W3, b3[None, :])


# speedup vs baseline: 1.9398x; 1.0067x over previous
import jax
import jax.numpy as jnp
import numpy as np
from jax.experimental import pallas as pl
from jax.experimental.pallas import tpu as pltpu

_H = 14
_LH = 10
_BB = 1024


def _fused(x_ref, w0_ref, gt_ref, w1v_ref, b1_ref, w2_ref,
           b2_ref, w3_ref, b3_ref, o_ref):
    H, LH = _H, _LH
    out_u, in_u = gt_ref.shape[1], gt_ref.shape[2]
    inv_in = np.float32(1.0 / (in_u - 1))
    inv_out = np.float32(1.0 / (out_u - 1))
    f32 = jnp.float32

    if True:
        gt = gt_ref[...]
        wT = jnp.transpose(w0_ref[...])

        csh = jnp.sum(gt, axis=2)
        rsh = jnp.sum(gt, axis=1)
        csw = jnp.sum(wT, axis=1, keepdims=True)
        rsw = jnp.sum(wT, axis=0, keepdims=True)

        A = w1v_ref[3:3 + H, :]
        B = w1v_ref[3 + H:3 + 2 * H, :]
        C = w1v_ref[3 + 2 * H:3 + 3 * H, :]
        meffT = jnp.transpose(A - B * inv_in - C * inv_out)

        dn = (((1,), (0,)), ((), ()))
        core = jax.lax.dot_general(meffT, gt, dn,
                                   preferred_element_type=f32)

        colcorr = (jnp.dot(jnp.transpose(B), csh,
                           preferred_element_type=f32) * inv_in
                   + jnp.transpose(csw * (w1v_ref[1:2, :] * inv_in)))
        rowcorr = (jnp.dot(jnp.transpose(C), rsh,
                           preferred_element_type=f32) * inv_out
                   + jnp.transpose(w1v_ref[2:3, :]) * (rsw * inv_out))
        weffT = jnp.transpose(w1v_ref[0:1, :] - w1v_ref[1:2, :] * inv_in
                              - w1v_ref[2:3, :] * inv_out)
        b1T = jnp.transpose(b1_ref[...])

        pre1 = (core + wT[None, :, :] * weffT[:, :, None]
                + colcorr[:, :, None] + rowcorr[:, None, :]
                + b1T[:, :, None])
        a1 = jnp.maximum(pre1, 0.0)

        pre2 = jax.lax.dot_general(jnp.transpose(w2_ref[...]), a1, dn,
                                   preferred_element_type=f32)
        a2 = jnp.maximum(pre2 + jnp.transpose(b2_ref[...])[:, :, None],
                         0.0)

        w3T = jnp.transpose(w3_ref[...][:, 0:1])
        updT = jax.lax.dot_general(w3T, a2, dn,
                                   preferred_element_type=f32)[0]
        new_wT = wT + updT + b3_ref[0, 0]
        new_w = jnp.transpose(new_wT)

    logits = jnp.maximum(
        jnp.dot(x_ref[...], new_w, preferred_element_type=f32), 0.0)
    m = jnp.max(logits, axis=1, keepdims=True)
    e = jnp.exp(logits - m)
    o_ref[...] = e / jnp.sum(e, axis=1, keepdims=True)


def kernel(X, w0, h0, W1, b1, W2, b2, W3, b3):
    in_u, out_u, H = h0.shape
    batch = X.shape[0]
    gt = h0.transpose(2, 1, 0)

    return pl.pallas_call(
        _fused,
        out_shape=jax.ShapeDtypeStruct((batch, out_u), jnp.float32),
    )(X, w0, gt, W1,
      b1[None, :], W2, b2[None, :], W3, b3[None, :])


# speedup vs baseline: 1.9530x; 1.0068x over previous
import jax
import jax.numpy as jnp
import numpy as np
from jax.experimental import pallas as pl
from jax.experimental.pallas import tpu as pltpu

_H = 14
_LH = 10
_BB = 1024


def _fused(x_ref, w0_ref, gt_ref, w1v_ref, b1_ref, w2_ref,
           b2_ref, w3_ref, b3_ref, o_ref):
    H, LH = _H, _LH
    out_u, in_u = gt_ref.shape[1], gt_ref.shape[2]
    inv_in = np.float32(1.0 / (in_u - 1))
    inv_out = np.float32(1.0 / (out_u - 1))
    f32 = jnp.float32

    if True:
        gt = gt_ref[...]
        wT = jnp.transpose(w0_ref[...])

        csh = jnp.sum(gt, axis=2)
        rsh = jnp.sum(gt, axis=1)
        csw = jnp.sum(wT, axis=1, keepdims=True)
        rsw = jnp.sum(wT, axis=0, keepdims=True)

        A = w1v_ref[3:3 + H, :]
        B = w1v_ref[3 + H:3 + 2 * H, :]
        C = w1v_ref[3 + 2 * H:3 + 3 * H, :]
        meffT = jnp.transpose(A - B * inv_in - C * inv_out)

        dn = (((1,), (0,)), ((), ()))
        core = jax.lax.dot_general(meffT, gt, dn,
                                   preferred_element_type=f32)

        colcorr = (jnp.dot(jnp.transpose(B), csh,
                           preferred_element_type=f32) * inv_in
                   + jnp.transpose(csw * (w1v_ref[1:2, :] * inv_in)))
        rowcorr = (jnp.dot(jnp.transpose(C), rsh,
                           preferred_element_type=f32) * inv_out
                   + jnp.transpose(w1v_ref[2:3, :]) * (rsw * inv_out))
        weffT = jnp.transpose(w1v_ref[0:1, :] - w1v_ref[1:2, :] * inv_in
                              - w1v_ref[2:3, :] * inv_out)
        b1T = jnp.transpose(b1_ref[...])

        pre1 = (core + wT[None, :, :] * weffT[:, :, None]
                + colcorr[:, :, None] + rowcorr[:, None, :]
                + b1T[:, :, None])
        a1 = jnp.maximum(pre1, 0.0)

        pre2 = jax.lax.dot_general(jnp.transpose(w2_ref[...]), a1, dn,
                                   preferred_element_type=f32)
        a2 = jnp.maximum(pre2 + jnp.transpose(b2_ref[...])[:, :, None],
                         0.0)

        w3T = jnp.transpose(w3_ref[...][:, 0:1])
        updT = jax.lax.dot_general(w3T, a2, dn,
                                   preferred_element_type=f32)[0]
        new_wT = wT + updT + b3_ref[0, 0]
        new_w = jnp.transpose(new_wT)

    logits = jnp.maximum(
        jnp.dot(x_ref[...], new_w, preferred_element_type=f32,
                precision=jax.lax.Precision.DEFAULT), 0.0)
    m = jnp.max(logits, axis=1, keepdims=True)
    e = jnp.exp(logits - m)
    o_ref[...] = e / jnp.sum(e, axis=1, keepdims=True)


def kernel(X, w0, h0, W1, b1, W2, b2, W3, b3):
    in_u, out_u, H = h0.shape
    batch = X.shape[0]
    gt = h0.transpose(2, 1, 0)

    return pl.pallas_call(
        _fused,
        out_shape=jax.ShapeDtypeStruct((batch, out_u), jnp.float32),
    )(X, w0, gt, W1,
      b1[None, :], W2, b2[None, :], W3, b3[None, :])
